# baseline (speedup 1.0000x reference)
import functools

import jax
import jax.numpy as jnp
from jax.experimental import pallas as pl

_LS = 0.1
_PAD_IDX = 0
_EPS = 1e-05
_TRG_FACTOR = 1.0 - _LS


def _biased_kl_body(pred_ref, tgt_ref, tval_ref, crow_ref, bt_ref, no_ref,
                    out_ref, *, n_biased):
    rows, vocab = pred_ref.shape
    cols = jax.lax.broadcasted_iota(jnp.int32, (rows, vocab), 1)
    pre = jnp.where(cols == tgt_ref[...], tval_ref[...], crow_ref[...])
    pre = jnp.where(cols == _PAD_IDX, _EPS, pre)
    bd = jnp.zeros((rows, vocab), jnp.float32)
    for k in range(n_biased):
        bd = jnp.where(cols == bt_ref[:, k:k + 1], no_ref[:, k:k + 1], bd)
    t = pre + bd
    out_ref[...] = t * (jnp.log(t) - pred_ref[...])


def kernel(pred, trg, biased_trg, biased_offset):
    b, s, v = pred.shape
    k = biased_trg.shape[-1]
    n = b * s
    base = _LS / (v - 2)

    pred2 = pred.reshape(n, v)
    tgt = trg.reshape(n, 1)
    pad = tgt == _PAD_IDX
    tval = (_TRG_FACTOR *
            (1.0 - biased_offset.reshape(-1)[(k - 1) * n:])).reshape(n, 1)
    tval = jnp.where(pad, _EPS, tval + _EPS)
    crow = jnp.where(pad, _EPS, base + _EPS)
    bt = biased_trg.reshape(n, k)
    no = jnp.where(pad, 0.0, (_TRG_FACTOR * biased_offset).reshape(n, k))

    block_rows = 256
    grid = (n // block_rows,)
    body = functools.partial(_biased_kl_body, n_biased=k)
    row_spec = lambda d: pl.BlockSpec((block_rows, d), lambda i: (i, 0))
    return pl.pallas_call(
        body,
        grid=grid,
        in_specs=[
            row_spec(v),
            row_spec(1),
            row_spec(1),
            row_spec(1),
            row_spec(k),
            row_spec(k),
        ],
        out_specs=row_spec(v),
        out_shape=jax.ShapeDtypeStruct((n, v), jnp.float32),
    )(pred2, tgt, tval, crow, bt, no)
